# Initial kernel scaffold; baseline (speedup 1.0000x reference)
import jax
import jax.numpy as jnp
from jax.experimental import pallas as pl


def kernel(x, weight):
    raise NotImplementedError("write your pallas kernel here")


# speedup vs baseline: 1.4995x; 1.4995x over previous
import functools

import jax
import jax.numpy as jnp
from jax import lax
from jax.experimental import pallas as pl
from jax.experimental.pallas import tpu as pltpu
from jax.experimental.pallas import tpu_sc as plsc

EMB_D = 32
CHUNK = 128
NC, NS = 2, 16
NW = NC * NS
K = 8
GROUP = CHUNK * K
NBUF = 2


def _build(B):
    steps_per_w = B // (NW * CHUNK)
    ngroups = steps_per_w // K
    npipe = (ngroups // NBUF) * NBUF
    assert B == NW * steps_per_w * CHUNK and steps_per_w % K == 0
    assert npipe >= 2 * NBUF

    mesh = plsc.VectorSubcoreMesh(core_axis_name="c", subcore_axis_name="s")

    @functools.partial(
        pl.kernel,
        mesh=mesh,
        compiler_params=pltpu.CompilerParams(use_tc_tiling_on_sc=False),
        out_type=jax.ShapeDtypeStruct((B, EMB_D), jnp.float32),
        scratch_types=[
            pltpu.VMEM((NBUF, K, CHUNK), jnp.int32),
            pltpu.VMEM((NBUF, GROUP, EMB_D), jnp.float32),
            pltpu.SemaphoreType.DMA,
            pltpu.SemaphoreType.DMA,
            pltpu.SemaphoreType.DMA,
            pltpu.SemaphoreType.DMA,
            pltpu.SemaphoreType.DMA,
            pltpu.SemaphoreType.DMA,
        ],
    )
    def emb(x_hbm, w_hbm, out_hbm, idxr, rows_v,
            gsem0, gsem1, wsem0, wsem1, isem0, isem1):
        gsem = (gsem0, gsem1)
        wsem = (wsem0, wsem1)
        isem = (isem0, isem1)
        wid = lax.axis_index("s") * NC + lax.axis_index("c")
        idx_base = wid * steps_per_w
        out_base = wid * steps_per_w * CHUNK

        def issue_gathers(b):
            for j in range(K):
                pltpu.async_copy(
                    w_hbm.at[idxr.at[b].at[j]],
                    rows_v.at[b].at[pl.ds(j * CHUNK, CHUNK)],
                    gsem[b])

        def drain_gathers(b):
            pltpu.make_async_copy(
                w_hbm.at[pl.ds(0, GROUP)], rows_v.at[b], gsem[b]).wait()

        for b in range(NBUF):
            pltpu.sync_copy(x_hbm.at[pl.ds(idx_base + b * K, K)], idxr.at[b])
            issue_gathers(b)

        def body(i, carry):
            for b in range(NBUF):
                g = NBUF * i + b
                drain_gathers(b)
                wcp = pltpu.async_copy(
                    rows_v.at[b],
                    out_hbm.at[pl.ds(out_base + g * GROUP, GROUP)],
                    wsem[b])
                icp = pltpu.async_copy(
                    x_hbm.at[pl.ds(idx_base + (g + NBUF) * K, K)],
                    idxr.at[b], isem[b])
                icp.wait()
                wcp.wait()
                issue_gathers(b)
            return carry

        lax.fori_loop(0, npipe // NBUF - 1, body, 0)

        for b in range(NBUF):
            g = npipe - NBUF + b
            drain_gathers(b)
            pltpu.sync_copy(
                rows_v.at[b],
                out_hbm.at[pl.ds(out_base + g * GROUP, GROUP)])

        for g in range(npipe, ngroups):
            pltpu.sync_copy(x_hbm.at[pl.ds(idx_base + g * K, K)], idxr.at[0])
            issue_gathers(0)
            drain_gathers(0)
            pltpu.sync_copy(
                rows_v.at[0],
                out_hbm.at[pl.ds(out_base + g * GROUP, GROUP)])

    return emb


def kernel(x, weight):
    b0, b1 = x.shape
    B = b0 * b1
    xf = x.astype(jnp.int32).reshape(B // CHUNK, CHUNK)
    out = _build(B)(xf, weight)
    return out.reshape(b0, b1, EMB_D)
